# baseline (speedup 1.0000x reference)
import functools

import jax
import jax.numpy as jnp
from jax import lax
from jax.experimental import pallas as pl
from jax.experimental.pallas import tpu as pltpu
from jax.experimental.pallas import tpu_sc as plsc

N = 10000
D = 128
H = 64
E = 320000

NC = 2
NS = 16
NW = NC * NS
CH = 128
PC = 80
NCHUNK = NW * PC
E_PAD = NCHUNK * CH
N_PAD = 10240
RPT = N_PAD // NS

_mesh = plsc.VectorSubcoreMesh(core_axis_name="c", subcore_axis_name="s")
_sc_params = pltpu.CompilerParams(use_tc_tiling_on_sc=False,
                                 disable_bounds_checks=True)


@functools.partial(
    pl.kernel,
    out_type=jax.ShapeDtypeStruct((NC, N_PAD, 8), jnp.float32),
    mesh=_mesh,
    compiler_params=_sc_params,
    scratch_types=[
        pltpu.VMEM((PC, CH), jnp.int32),
        pltpu.VMEM((CH, 8), jnp.float32),
        pltpu.VMEM_SHARED((N_PAD, 8), jnp.float32),
    ],
)
def _sc_degree(dst_hbm, ones_hbm, zeros_hbm, out_hbm, dst_v, ones_v, acc):
    cid = lax.axis_index("c")
    sid = lax.axis_index("s")
    gwid = cid * NS + sid
    pltpu.sync_copy(zeros_hbm, acc.at[pl.ds(sid * RPT, RPT)])
    pltpu.sync_copy(dst_hbm.at[pl.ds(gwid * PC, PC)], dst_v)
    pltpu.sync_copy(ones_hbm, ones_v)
    plsc.subcore_barrier()

    def chunk(j, carry):
        pltpu.sync_copy(ones_v, acc.at[dst_v.at[j]], add=True)
        return carry

    lax.fori_loop(0, PC, chunk, 0, unroll=False)
    plsc.subcore_barrier()
    pltpu.sync_copy(acc.at[pl.ds(sid * RPT, RPT)],
                    out_hbm.at[cid, pl.ds(sid * RPT, RPT)])


NB = 4
NG = PC // NB


@functools.partial(
    pl.kernel,
    out_type=jax.ShapeDtypeStruct((NC, N_PAD, H), jnp.float32),
    mesh=_mesh,
    compiler_params=_sc_params,
    scratch_types=[
        pltpu.VMEM((PC, CH), jnp.int32),
        pltpu.VMEM((PC, CH), jnp.int32),
        [pltpu.VMEM((CH, H), jnp.float32)] * NB,
        pltpu.VMEM_SHARED((N_PAD, H), jnp.float32),
        [pltpu.SemaphoreType.DMA] * NB,
    ],
)
def _sc_scatter(y_hbm, src_hbm, dst_hbm, zeros_hbm, out_hbm,
                src_v, dst_v, rows, acc, sems):
    cid = lax.axis_index("c")
    sid = lax.axis_index("s")
    gwid = cid * NS + sid
    pltpu.sync_copy(zeros_hbm, acc.at[pl.ds(sid * RPT, RPT)])
    pltpu.sync_copy(src_hbm.at[pl.ds(gwid * PC, PC)], src_v)
    pltpu.sync_copy(dst_hbm.at[pl.ds(gwid * PC, PC)], dst_v)
    plsc.subcore_barrier()

    for b in range(NB):
        pltpu.async_copy(y_hbm.at[src_v.at[b]], rows[b], sems[b])

    def group(g, carry):
        for b in range(NB):
            j = g * NB + b
            pltpu.make_async_copy(y_hbm.at[src_v.at[j]], rows[b],
                                  sems[b]).wait()
            pltpu.sync_copy(rows[b], acc.at[dst_v.at[j]], add=True)

            @pl.when(g < NG - 1)
            def _prefetch(b=b, j=j):
                pltpu.async_copy(y_hbm.at[src_v.at[j + NB]], rows[b], sems[b])
        return carry

    lax.fori_loop(0, NG, group, 0, unroll=False)
    plsc.subcore_barrier()
    pltpu.sync_copy(acc.at[pl.ds(sid * RPT, RPT)],
                    out_hbm.at[cid, pl.ds(sid * RPT, RPT)])


def _tc_mm_body(x_ref, w_ref, xw_ref):
    xw_ref[...] = jnp.dot(
        x_ref[...], w_ref[...], preferred_element_type=jnp.float32)


def _tc_bn_body(t_ref, g_ref, be_ref, w_ref, xw_ref):
    t = t_ref[...]
    mean = jnp.mean(t, axis=0, keepdims=True)
    c = t - mean
    var = jnp.mean(c * c, axis=0, keepdims=True)
    h = jnp.maximum(c * lax.rsqrt(var + 1e-5) * g_ref[...] + be_ref[...], 0.0)
    xw_ref[...] = jnp.dot(h, w_ref[...], preferred_element_type=jnp.float32)


def _tc_fc_body(t_ref, fcw_ref, fcb_ref, out_ref):
    h = jnp.maximum(t_ref[...], 0.0)
    out_ref[...] = jnp.dot(
        h, fcw_ref[...], preferred_element_type=jnp.float32) + fcb_ref[...]


_tc_mm = pl.pallas_call(
    _tc_mm_body,
    out_shape=jax.ShapeDtypeStruct((N, H), jnp.float32),
)

_tc_bn = pl.pallas_call(
    _tc_bn_body,
    out_shape=jax.ShapeDtypeStruct((N, H), jnp.float32),
)

_tc_fc = pl.pallas_call(
    _tc_fc_body,
    out_shape=jax.ShapeDtypeStruct((N, 2), jnp.float32),
)


def kernel(x, edge_index, edge_attr, W1, b1, W2, b2, W3, b3, g1, be1, g2, be2,
           fcW, fcb):
    ei = edge_index.astype(jnp.int32)
    pad = E_PAD - E
    pad_src = jnp.arange(pad, dtype=jnp.int32) % N
    pad_dst = N + jnp.arange(pad, dtype=jnp.int32) % (N_PAD - N)
    src2d = jnp.concatenate([ei[0], pad_src]).reshape(NCHUNK, CH)
    dst2d = jnp.concatenate([ei[1], pad_dst]).reshape(NCHUNK, CH)

    ones1 = jnp.ones((CH, 8), jnp.float32)
    zeros1 = jnp.zeros((RPT, 8), jnp.float32)
    zerosH = jnp.zeros((RPT, H), jnp.float32)

    cnt = _sc_degree(dst2d, ones1, zeros1)
    xw1 = _tc_mm(x, W1)

    dinv = lax.rsqrt(cnt[0, :N, 0:1] + cnt[1, :N, 0:1] + 1.0)

    y1 = xw1 * dinv
    acc1 = _sc_scatter(y1, src2d, dst2d, zerosH)
    t1 = (acc1[0, :N] + acc1[1, :N] + y1) * dinv + b1
    y2 = _tc_bn(t1, g1.reshape(1, H), be1.reshape(1, H), W2) * dinv
    acc2 = _sc_scatter(y2, src2d, dst2d, zerosH)
    t2 = (acc2[0, :N] + acc2[1, :N] + y2) * dinv + b2
    y3 = _tc_bn(t2, g2.reshape(1, H), be2.reshape(1, H), W3) * dinv
    acc3 = _sc_scatter(y3, src2d, dst2d, zerosH)
    t3 = (acc3[0, :N] + acc3[1, :N] + y3) * dinv + b3
    return _tc_fc(t3, fcW, fcb.reshape(1, 2))


# speedup vs baseline: 1.1072x; 1.1072x over previous
import functools

import jax
import jax.numpy as jnp
from jax import lax
from jax.experimental import pallas as pl
from jax.experimental.pallas import tpu as pltpu
from jax.experimental.pallas import tpu_sc as plsc

N = 10000
D = 128
H = 64
E = 320000

NC = 2
NS = 16
NW = NC * NS
CH = 128
PC = 80
NCHUNK = NW * PC
E_PAD = NCHUNK * CH
N_PAD = 10240
RPT = N_PAD // NS

_mesh = plsc.VectorSubcoreMesh(core_axis_name="c", subcore_axis_name="s")
_sc_params = pltpu.CompilerParams(use_tc_tiling_on_sc=False,
                                 disable_bounds_checks=True)


@functools.partial(
    pl.kernel,
    out_type=jax.ShapeDtypeStruct((NC, N_PAD, 8), jnp.float32),
    mesh=_mesh,
    compiler_params=_sc_params,
    scratch_types=[
        pltpu.VMEM((PC, CH), jnp.int32),
        pltpu.VMEM((CH, 8), jnp.float32),
        pltpu.VMEM_SHARED((N_PAD, 8), jnp.float32),
    ],
)
def _sc_degree(dst_hbm, ones_hbm, zeros_hbm, out_hbm, dst_v, ones_v, acc):
    cid = lax.axis_index("c")
    sid = lax.axis_index("s")
    gwid = cid * NS + sid
    pltpu.sync_copy(zeros_hbm, acc.at[pl.ds(sid * RPT, RPT)])
    pltpu.sync_copy(dst_hbm.at[pl.ds(gwid * PC, PC)], dst_v)
    pltpu.sync_copy(ones_hbm, ones_v)
    plsc.subcore_barrier()

    def chunk(j, carry):
        pltpu.sync_copy(ones_v, acc.at[dst_v.at[j]], add=True)
        return carry

    lax.fori_loop(0, PC, chunk, 0, unroll=False)
    plsc.subcore_barrier()
    pltpu.sync_copy(acc.at[pl.ds(sid * RPT, RPT)],
                    out_hbm.at[cid, pl.ds(sid * RPT, RPT)])


NB = 4
NG = PC // NB


@functools.partial(
    pl.kernel,
    out_type=jax.ShapeDtypeStruct((NC, N_PAD, H), jnp.float32),
    mesh=_mesh,
    compiler_params=_sc_params,
    scratch_types=[
        pltpu.VMEM((PC, CH), jnp.int32),
        pltpu.VMEM((PC, CH), jnp.int32),
        [pltpu.VMEM((CH, H), jnp.float32)] * NB,
        pltpu.VMEM_SHARED((N_PAD, H), jnp.float32),
        [pltpu.SemaphoreType.DMA] * NB,
    ],
)
def _sc_scatter(y_hbm, src_hbm, dst_hbm, zeros_hbm, out_hbm,
                src_v, dst_v, rows, acc, sems):
    cid = lax.axis_index("c")
    sid = lax.axis_index("s")
    gwid = cid * NS + sid
    pltpu.sync_copy(zeros_hbm, acc.at[pl.ds(sid * RPT, RPT)])
    pltpu.sync_copy(src_hbm.at[pl.ds(gwid * PC, PC)], src_v)
    pltpu.sync_copy(dst_hbm.at[pl.ds(gwid * PC, PC)], dst_v)
    plsc.subcore_barrier()

    for b in range(NB):
        pltpu.async_copy(y_hbm.at[src_v.at[b]], rows[b], sems[b])

    def group(g, carry):
        for b in range(NB):
            j = g * NB + b
            pltpu.make_async_copy(y_hbm.at[src_v.at[j]], rows[b],
                                  sems[b]).wait()
            pltpu.sync_copy(rows[b], acc.at[dst_v.at[j]], add=True)

            @pl.when(g < NG - 1)
            def _prefetch(b=b, j=j):
                pltpu.async_copy(y_hbm.at[src_v.at[j + NB]], rows[b], sems[b])
        return carry

    lax.fori_loop(0, NG, group, 0, unroll=False)
    plsc.subcore_barrier()
    pltpu.sync_copy(acc.at[pl.ds(sid * RPT, RPT)],
                    out_hbm.at[cid, pl.ds(sid * RPT, RPT)])


def _tc_mm_body(x_ref, w_ref, xw_ref):
    xw_ref[...] = jnp.dot(
        x_ref[...], w_ref[...], preferred_element_type=jnp.float32)


def _tc_deg_body(cnt_ref, xw_ref, dinv_ref, y_ref):
    dinv = lax.rsqrt(cnt_ref[0, :N, 0:1] + cnt_ref[1, :N, 0:1] + 1.0)
    dinv_ref[...] = dinv
    y_ref[...] = xw_ref[...] * dinv


def _tc_bn_body(acc_ref, y_ref, dinv_ref, b_ref, g_ref, be_ref, w_ref,
                yn_ref):
    dinv = dinv_ref[...]
    t = ((acc_ref[0, :N] + acc_ref[1, :N] + y_ref[...]) * dinv + b_ref[...])
    mean = jnp.mean(t, axis=0, keepdims=True)
    c = t - mean
    var = jnp.mean(c * c, axis=0, keepdims=True)
    h = jnp.maximum(c * lax.rsqrt(var + 1e-5) * g_ref[...] + be_ref[...], 0.0)
    yn_ref[...] = jnp.dot(
        h, w_ref[...], preferred_element_type=jnp.float32) * dinv


def _tc_fc_body(acc_ref, y_ref, dinv_ref, b_ref, fcw_ref, fcb_ref, out_ref):
    t = ((acc_ref[0, :N] + acc_ref[1, :N] + y_ref[...]) * dinv_ref[...]
         + b_ref[...])
    h = jnp.maximum(t, 0.0)
    out_ref[...] = jnp.dot(
        h, fcw_ref[...], preferred_element_type=jnp.float32) + fcb_ref[...]


_tc_mm = pl.pallas_call(
    _tc_mm_body,
    out_shape=jax.ShapeDtypeStruct((N, H), jnp.float32),
)

_tc_deg = pl.pallas_call(
    _tc_deg_body,
    out_shape=(jax.ShapeDtypeStruct((N, 1), jnp.float32),
               jax.ShapeDtypeStruct((N, H), jnp.float32)),
)

_tc_bn = pl.pallas_call(
    _tc_bn_body,
    out_shape=jax.ShapeDtypeStruct((N, H), jnp.float32),
)

_tc_fc = pl.pallas_call(
    _tc_fc_body,
    out_shape=jax.ShapeDtypeStruct((N, 2), jnp.float32),
)


def kernel(x, edge_index, edge_attr, W1, b1, W2, b2, W3, b3, g1, be1, g2, be2,
           fcW, fcb):
    ei = edge_index.astype(jnp.int32)
    pad = E_PAD - E
    pad_src = jnp.arange(pad, dtype=jnp.int32) % N
    pad_dst = N + jnp.arange(pad, dtype=jnp.int32) % (N_PAD - N)
    src2d = jnp.concatenate([ei[0], pad_src]).reshape(NCHUNK, CH)
    dst2d = jnp.concatenate([ei[1], pad_dst]).reshape(NCHUNK, CH)

    ones1 = jnp.ones((CH, 8), jnp.float32)
    zeros1 = jnp.zeros((RPT, 8), jnp.float32)
    zerosH = jnp.zeros((RPT, H), jnp.float32)

    cnt = _sc_degree(dst2d, ones1, zeros1)
    xw1 = _tc_mm(x, W1)

    dinv, y1 = _tc_deg(cnt, xw1)
    acc1 = _sc_scatter(y1, src2d, dst2d, zerosH)
    y2 = _tc_bn(acc1, y1, dinv, b1.reshape(1, H), g1.reshape(1, H),
                be1.reshape(1, H), W2)
    acc2 = _sc_scatter(y2, src2d, dst2d, zerosH)
    y3 = _tc_bn(acc2, y2, dinv, b2.reshape(1, H), g2.reshape(1, H),
                be2.reshape(1, H), W3)
    acc3 = _sc_scatter(y3, src2d, dst2d, zerosH)
    return _tc_fc(acc3, y3, dinv, b3.reshape(1, H), fcW, fcb.reshape(1, 2))


# speedup vs baseline: 1.3288x; 1.2002x over previous
import functools

import jax
import jax.numpy as jnp
from jax import lax
from jax.experimental import pallas as pl
from jax.experimental.pallas import tpu as pltpu
from jax.experimental.pallas import tpu_sc as plsc

N = 10000
D = 128
H = 64
E = 320000

NC = 2
NS = 16
NW = NC * NS
CH = 128
PC = 80
NCHUNK = NW * PC
E_PAD = NCHUNK * CH
N_PAD = 10240
RPT = N_PAD // NS

_mesh = plsc.VectorSubcoreMesh(core_axis_name="c", subcore_axis_name="s")
_sc_params = pltpu.CompilerParams(use_tc_tiling_on_sc=False,
                                 disable_bounds_checks=True)


@functools.partial(
    pl.kernel,
    out_type=jax.ShapeDtypeStruct((NC, N_PAD, 8), jnp.float32),
    mesh=_mesh,
    compiler_params=_sc_params,
    scratch_types=[
        pltpu.VMEM((PC, CH), jnp.int32),
        pltpu.VMEM((CH, 8), jnp.float32),
        pltpu.VMEM_SHARED((N_PAD, 8), jnp.float32),
    ],
)
def _sc_degree(dst_hbm, ones_hbm, zeros_hbm, out_hbm, dst_v, ones_v, acc):
    cid = lax.axis_index("c")
    sid = lax.axis_index("s")
    gwid = cid * NS + sid
    pltpu.sync_copy(zeros_hbm, acc.at[pl.ds(sid * RPT, RPT)])
    pltpu.sync_copy(dst_hbm.at[pl.ds(gwid * PC, PC)], dst_v)
    pltpu.sync_copy(ones_hbm, ones_v)
    plsc.subcore_barrier()

    def chunk(j, carry):
        pltpu.sync_copy(ones_v, acc.at[dst_v.at[j]], add=True)
        return carry

    lax.fori_loop(0, PC, chunk, 0, unroll=False)
    plsc.subcore_barrier()
    pltpu.sync_copy(acc.at[pl.ds(sid * RPT, RPT)],
                    out_hbm.at[cid, pl.ds(sid * RPT, RPT)])


NB = 4
NG = PC // NB


@functools.partial(
    pl.kernel,
    out_type=jax.ShapeDtypeStruct((NC, N_PAD, H), jnp.float32),
    mesh=_mesh,
    compiler_params=_sc_params,
    scratch_types=[
        pltpu.VMEM((PC, CH), jnp.int32),
        pltpu.VMEM((PC, CH), jnp.int32),
        [pltpu.VMEM((CH, H), jnp.float32)] * NB,
        pltpu.VMEM_SHARED((N_PAD, H), jnp.float32),
        [pltpu.SemaphoreType.DMA] * NB,
    ],
)
def _sc_scatter(y_hbm, src_hbm, dst_hbm, zeros_hbm, out_hbm,
                src_v, dst_v, rows, acc, sems):
    cid = lax.axis_index("c")
    sid = lax.axis_index("s")
    gwid = cid * NS + sid
    pltpu.sync_copy(zeros_hbm, acc.at[pl.ds(sid * RPT, RPT)])
    pltpu.sync_copy(src_hbm.at[pl.ds(gwid * PC, PC)], src_v)
    pltpu.sync_copy(dst_hbm.at[pl.ds(gwid * PC, PC)], dst_v)
    plsc.subcore_barrier()

    for b in range(NB):
        pltpu.async_copy(y_hbm.at[src_v.at[b]], rows[b], sems[b])

    def group(g, carry):
        for b in range(NB):
            j = g * NB + b
            pltpu.make_async_copy(y_hbm.at[src_v.at[j]], rows[b],
                                  sems[b]).wait()
            pltpu.sync_copy(rows[b], acc.at[dst_v.at[j]], add=True)

            @pl.when(g < NG - 1)
            def _prefetch(b=b, j=j):
                pltpu.async_copy(y_hbm.at[src_v.at[j + NB]], rows[b], sems[b])
        return carry

    lax.fori_loop(0, NG, group, 0, unroll=False)
    plsc.subcore_barrier()
    pltpu.sync_copy(acc.at[pl.ds(sid * RPT, RPT)],
                    out_hbm.at[cid, pl.ds(sid * RPT, RPT)])


NF = N // 2


def _tc_mm_body(x_ref, w_ref, xw_ref):
    xw_ref[...] = jnp.dot(
        x_ref[...], w_ref[...], preferred_element_type=jnp.float32)


def _tc_bn_body(acc_ref, y_ref, dinv_ref, b_ref, g_ref, be_ref, w_ref,
                yn_ref):
    dinv = dinv_ref[...]
    t = ((acc_ref[0, :NF] + acc_ref[1, :NF] + y_ref[...]) * dinv + b_ref[...])
    cs = jnp.sum(t, axis=0, keepdims=True)
    m = (cs[:, :H] + cs[:, H:]) * (1.0 / N)
    mb = jnp.concatenate([m, m], axis=1)
    c = t - mb
    vs = jnp.sum(c * c, axis=0, keepdims=True)
    v = (vs[:, :H] + vs[:, H:]) * (1.0 / N)
    vb = jnp.concatenate([v, v], axis=1)
    h = jnp.maximum(c * lax.rsqrt(vb + 1e-5) * g_ref[...] + be_ref[...], 0.0)
    yn_ref[...] = jnp.dot(
        h, w_ref[...], preferred_element_type=jnp.float32) * dinv


def _tc_fc_body(acc_ref, y_ref, dinv_ref, b_ref, fcw_ref, fcb_ref, out_ref):
    t = ((acc_ref[0, :NF] + acc_ref[1, :NF] + y_ref[...]) * dinv_ref[...]
         + b_ref[...])
    h = jnp.maximum(t, 0.0)
    out_ref[...] = jnp.dot(
        h, fcw_ref[...], preferred_element_type=jnp.float32) + fcb_ref[...]


_tc_mm = pl.pallas_call(
    _tc_mm_body,
    out_shape=jax.ShapeDtypeStruct((N, H), jnp.float32),
)

_tc_bn = pl.pallas_call(
    _tc_bn_body,
    out_shape=jax.ShapeDtypeStruct((NF, 2 * H), jnp.float32),
)

_tc_fc = pl.pallas_call(
    _tc_fc_body,
    out_shape=jax.ShapeDtypeStruct((NF, 4), jnp.float32),
)


def _dup(p):
    return jnp.concatenate([p.reshape(1, -1), p.reshape(1, -1)], axis=1)


def _blockdiag(w):
    k, m = w.shape
    z = jnp.zeros((k, m), w.dtype)
    return jnp.concatenate(
        [jnp.concatenate([w, z], axis=1), jnp.concatenate([z, w], axis=1)],
        axis=0)


def kernel(x, edge_index, edge_attr, W1, b1, W2, b2, W3, b3, g1, be1, g2, be2,
           fcW, fcb):
    ei = edge_index.astype(jnp.int32)
    pad = E_PAD - E
    pad_src = jnp.arange(pad, dtype=jnp.int32) % N
    pad_dst = N + jnp.arange(pad, dtype=jnp.int32) % (N_PAD - N)
    src2d = jnp.concatenate([ei[0], pad_src]).reshape(NCHUNK, CH)
    dst2d = jnp.concatenate([ei[1], pad_dst]).reshape(NCHUNK, CH)

    ones1 = jnp.ones((CH, 8), jnp.float32)
    zeros1 = jnp.zeros((RPT, 8), jnp.float32)
    zerosH = jnp.zeros((RPT, H), jnp.float32)

    cnt = _sc_degree(dst2d, ones1, zeros1)
    xw1 = _tc_mm(x, W1)

    dinv = lax.rsqrt(cnt[0, :N, 0:1] + cnt[1, :N, 0:1] + 1.0)
    dinv_f = jnp.broadcast_to(dinv.reshape(NF, 2, 1),
                              (NF, 2, H)).reshape(NF, 2 * H)
    y1_f = (xw1 * dinv).reshape(NF, 2 * H)

    W2d, W3d = _blockdiag(W2), _blockdiag(W3)
    fcWd = _blockdiag(fcW)

    acc1 = _sc_scatter(y1_f.reshape(N, H), src2d, dst2d, zerosH)
    y2_f = _tc_bn(acc1.reshape(NC, N_PAD // 2, 2 * H), y1_f, dinv_f,
                  _dup(b1), _dup(g1), _dup(be1), W2d)
    acc2 = _sc_scatter(y2_f.reshape(N, H), src2d, dst2d, zerosH)
    y3_f = _tc_bn(acc2.reshape(NC, N_PAD // 2, 2 * H), y2_f, dinv_f,
                  _dup(b2), _dup(g2), _dup(be2), W3d)
    acc3 = _sc_scatter(y3_f.reshape(N, H), src2d, dst2d, zerosH)
    out_f = _tc_fc(acc3.reshape(NC, N_PAD // 2, 2 * H), y3_f, dinv_f,
                   _dup(b3), fcWd, _dup(fcb))
    return out_f.reshape(N, 2)
